# Initial kernel scaffold; baseline (speedup 1.0000x reference)
import jax
import jax.numpy as jnp
from jax.experimental import pallas as pl


def kernel(heads, head_embeds, tails, weights):
    raise NotImplementedError("write your pallas kernel here")


# speedup vs baseline: 1.1258x; 1.1258x over previous
import functools

import jax
import jax.numpy as jnp
import numpy as np
from jax import lax
from jax.experimental import pallas as pl
from jax.experimental.pallas import tpu as pltpu
from jax.experimental.pallas import tpu_sc as plsc

B = 16384
D = 64
NNEG = 64
NGENES = 1000000
NW = 32
NB = B // NW
CB = 2
NCH = NB // CB
L = 16
NBLK = NGENES // 128
NSLAB = NGENES // 2

_cache = {}

_PARAMS = pltpu.CompilerParams(
    needs_layout_passes=False, use_tc_tiling_on_sc=True)
_MESH = plsc.VectorSubcoreMesh(core_axis_name="c", subcore_axis_name="s")


def _neg_idx3d():
    if "neg" not in _cache:
        try:
            with jax.ensure_compile_time_eval(), \
                    jax.default_device(jax.devices("cpu")[0]):
                nt = jax.random.randint(jax.random.key(42), (B, NNEG), 0, NGENES)
                nt = np.asarray(nt, np.int32).reshape(NW, (NB * NNEG) // 128, 128)
            _cache["neg"] = nt
        except Exception:
            nt = jax.random.randint(jax.random.key(42), (B, NNEG), 0, NGENES)
            return jnp.asarray(nt, jnp.int32).reshape(NW, (NB * NNEG) // 128, 128)
    return _cache["neg"]


def _logsig(s):
    u = jnp.exp(-jnp.abs(s))
    t = u / (u + 2.0)
    t2 = t * t
    p = 1.0 + t2 * (1.0 / 3.0 + t2 * (0.2 + t2 * (1.0 / 7.0)))
    return jnp.minimum(s, 0.0) - 2.0 * t * p


@functools.partial(
    pl.kernel,
    out_type=jax.ShapeDtypeStruct((NSLAB, 128), jnp.float32),
    mesh=_MESH,
    compiler_params=_PARAMS,
    scratch_types=[
        pltpu.VMEM((64, 128), jnp.float32),
        pltpu.VMEM((64, 128), jnp.float32),
        pltpu.VMEM((64, 128), jnp.float32),
        pltpu.VMEM((64, 128), jnp.float32),
        pltpu.VMEM((64, 64), jnp.float32),
        pltpu.VMEM((32, 128), jnp.float32),
        pltpu.SemaphoreType.DMA,
        pltpu.SemaphoreType.DMA,
        pltpu.SemaphoreType.DMA,
        pltpu.SemaphoreType.DMA,
    ],
)
def _sc_transpose(wt_hbm, out_hbm, bin0, bin1, bout0, bout1, tin, tout,
                  si0, si1, so0, so1):
    wid = lax.axis_index("s") * 2 + lax.axis_index("c")
    bins = (bin0, bin1)
    bouts = (bout0, bout1)
    sis = (si0, si1)
    sos = (so0, so1)
    iotav = lax.iota(jnp.int32, L)

    def blk_of(k):
        return jnp.minimum(wid + k * NW, NBLK - 1)

    def issue_in(k, p):
        i0 = blk_of(k) * 128
        pltpu.async_copy(wt_hbm.at[:, pl.ds(i0, 128)], bins[p], sis[p])

    def drain_in(p):
        pltpu.make_async_copy(wt_hbm.at[:, pl.ds(0, 128)], bins[p], sis[p]).wait()

    def issue_out(k, p):
        r0 = blk_of(k) * 64
        pltpu.async_copy(bouts[p], out_hbm.at[pl.ds(r0, 64), :], sos[p])

    def drain_out(p):
        pltpu.make_async_copy(bouts[p], out_hbm.at[pl.ds(0, 64), :], sos[p]).wait()

    def transpose_block(p):
        def body(r, _):
            t = (r + iotav) & 127
            rs = t >> 1
            low = (t & 1) << 6
            for jb in range(4):
                rowv = iotav + (jb * L)
                v = plsc.load_gather(bins[p], [rowv, t])
                cs = low + rowv
                plsc.store_scatter(bouts[p], [rs, cs], v)
            return 0
        lax.fori_loop(0, 128, body, 0)

    issue_in(0, 0)
    issue_in(1, 1)

    def outer(k2, _):
        for p in (0, 1):
            k = k2 * 2 + p
            drain_in(p)

            @pl.when(k >= 2)
            def _():
                drain_out(p)

            transpose_block(p)
            issue_out(k, p)
            issue_in(k + 2, p)
        return 0

    lax.fori_loop(0, 122, outer, 0)

    drain_in(0)
    drain_out(0)

    @pl.when(wid <= 3)
    def _():
        transpose_block(0)
        issue_out(244, 0)

    drain_in(1)
    drain_out(1)

    @pl.when(wid <= 3)
    def _():
        drain_out(0)

    @pl.when(wid == 31)
    def _():
        pltpu.sync_copy(wt_hbm.at[:, pl.ds(NBLK * 128, 64)], tin)

        def body(r, _):
            t = (r + iotav) & 63
            rs = t >> 1
            low = (t & 1) << 6
            for jb in range(4):
                rowv = iotav + (jb * L)
                v = plsc.load_gather(tin, [rowv, t])
                cs = low + rowv
                plsc.store_scatter(tout, [rs, cs], v)
            return 0
        lax.fori_loop(0, 64, body, 0)
        pltpu.sync_copy(tout, out_hbm.at[pl.ds(NBLK * 64, 32), :])


@functools.partial(
    pl.kernel,
    out_type=jax.ShapeDtypeStruct((NW, L), jnp.float32),
    mesh=_MESH,
    compiler_params=_PARAMS,
    scratch_types=[
        pltpu.VMEM((NCH, 128), jnp.int32),
        pltpu.VMEM((NCH, 128), jnp.int32),
        pltpu.VMEM((4, 128), jnp.int32),
        pltpu.VMEM((4, 128), jnp.int32),
        pltpu.VMEM((128, 128), jnp.float32),
        pltpu.VMEM((128, 128), jnp.float32),
        pltpu.VMEM((128,), jnp.float32),
        pltpu.VMEM((128,), jnp.float32),
        pltpu.VMEM((64, 128), jnp.float32),
        pltpu.VMEM((L,), jnp.float32),
        pltpu.SemaphoreType.DMA,
        pltpu.SemaphoreType.DMA,
        pltpu.SemaphoreType.DMA,
        pltpu.SemaphoreType.DMA,
    ],
)
def _sc_loss(hs_hbm, tails_hbm, neg_hbm, w_hbm, out_hbm,
             idx_v, slab_v, traw_v, tslab_v, rows0, rows1, hb0, hb1,
             hpos, out_v, sr0, sr1, sh0, sh1):
    wid = lax.axis_index("s") * 2 + lax.axis_index("c")
    rows = (rows0, rows1)
    srs = (sr0, sr1)
    hbs = (hb0, hb1)
    shs = (sh0, sh1)
    iotav = lax.iota(jnp.int32, L)
    zero16 = jnp.zeros((L,), jnp.float32)

    pltpu.sync_copy(neg_hbm.at[wid], idx_v)
    pltpu.sync_copy(tails_hbm.at[wid], traw_v)

    def conv(t, _):
        r = t >> 3
        c = (t & 7) * L
        slab_v[r, pl.ds(c, L)] = idx_v[r, pl.ds(c, L)] >> 1
        return 0
    lax.fori_loop(0, NCH * 8, conv, 0)
    for t in range(32):
        r, c = t // 8, (t % 8) * L
        tslab_v[r, pl.ds(c, L)] = traw_v[r, pl.ds(c, L)] >> 1

    def issue(c, p):
        pltpu.async_copy(w_hbm.at[slab_v.at[c]], rows[p], srs[p])
        pltpu.async_copy(hs_hbm.at[wid * NCH + c], hbs[p], shs[p])

    def drain(p):
        pltpu.make_async_copy(w_hbm.at[pl.ds(0, 128)], rows[p], srs[p]).wait()
        pltpu.make_async_copy(hs_hbm.at[0], hbs[p], shs[p]).wait()

    issue(0, 0)
    issue(1, 1)

    rowbase = [iotav + k * L for k in range(8)]

    def chunk_compute(c, p, loss):
        paroffs = [(idx_v[c, pl.ds(k * L, L)] & 1) << 6 for k in range(8)]

        def dbody(dd, accs):
            t = (iotav + dd) & (D - 1)
            new = []
            for b in range(CB):
                hd = plsc.load_gather(hbs[p], [(b * D) + t])
                for g in range(4):
                    k = b * 4 + g
                    wv = plsc.load_gather(rows[p], [rowbase[k], paroffs[k] + t])
                    new.append(accs[k] + wv * hd)
            return tuple(new)
        accs = lax.fori_loop(0, D, dbody, tuple(zero16 for _ in range(8)))
        for a in accs:
            loss = loss + _logsig(-a)
        return loss

    def outer(c2, loss):
        for p in (0, 1):
            c = c2 * 2 + p
            drain(p)
            loss = chunk_compute(c, p, loss)
            nc = c + 2
            nc = jnp.where(nc >= NCH, nc - NCH, nc)
            issue(nc, p)
        return loss

    loss = lax.fori_loop(0, NCH // 2, outer, zero16)
    drain(0)
    drain(1)

    for sp in range(4):
        pltpu.async_copy(w_hbm.at[tslab_v.at[sp]], rows0, sr0)
        pltpu.async_copy(
            hs_hbm.at[pl.ds(wid * NCH + sp * 64, 64), :], hpos, sh0)
        pltpu.make_async_copy(w_hbm.at[pl.ds(0, 128)], rows0, sr0).wait()
        pltpu.make_async_copy(
            hs_hbm.at[pl.ds(0, 64), :], hpos, sh0).wait()
        for bg in range(8):
            par = (traw_v[sp, pl.ds(bg * L, L)] & 1) << 6
            rb = iotav + bg * L
            hrow = rb >> 1
            hlow = (rb & 1) << 6

            def dbody(dd, acc):
                t = (iotav + dd) & (D - 1)
                wv = plsc.load_gather(rows0, [rb, par + t])
                hv = plsc.load_gather(hpos, [hrow, hlow + t])
                return acc + wv * hv
            s = lax.fori_loop(0, D, dbody, zero16)
            loss = loss + _logsig(s)

    out_v[...] = loss
    pltpu.sync_copy(out_v, out_hbm.at[wid])


def kernel(heads, head_embeds, tails, weights):
    del heads
    neg = _neg_idx3d()
    tails3 = tails.astype(jnp.int32).reshape(NW, 4, 128)
    hslab = head_embeds.reshape(B // 2, 128)
    wlin = _sc_transpose(weights.T)
    part = _sc_loss(hslab, tails3, neg, wlin)
    return -(jnp.sum(part) / B)


# speedup vs baseline: 1.3395x; 1.1898x over previous
import functools

import jax
import jax.numpy as jnp
import numpy as np
from jax import lax
from jax.experimental import pallas as pl
from jax.experimental.pallas import tpu as pltpu
from jax.experimental.pallas import tpu_sc as plsc

B = 16384
D = 64
NNEG = 64
NGENES = 1000000
NW = 32
NB = B // NW
CB = 4
NCH = NB // CB
ROWS = CB * NNEG
L = 16
NBLK = NGENES // 128
NSLAB = NGENES // 2

_cache = {}
_MESH = plsc.VectorSubcoreMesh(core_axis_name="c", subcore_axis_name="s")


def _neg_idx3d():
    if "neg" not in _cache:
        try:
            with jax.ensure_compile_time_eval(), \
                    jax.default_device(jax.devices("cpu")[0]):
                nt = jax.random.randint(jax.random.key(42), (B, NNEG), 0, NGENES)
                nt = np.asarray(nt, np.int32).reshape(NW, (NB * NNEG) // 128, 128)
            _cache["neg"] = nt
        except Exception:
            nt = jax.random.randint(jax.random.key(42), (B, NNEG), 0, NGENES)
            return jnp.asarray(nt, jnp.int32).reshape(NW, (NB * NNEG) // 128, 128)
    return _cache["neg"]


def _logsig(s):
    u = jnp.exp(-jnp.abs(s))
    t = u / (u + 2.0)
    t2 = t * t
    p = 1.0 + t2 * (1.0 / 3.0 + t2 * (0.2 + t2 * (1.0 / 7.0)))
    return jnp.minimum(s, 0.0) - 2.0 * t * p


@functools.partial(
    pl.kernel,
    out_type=jax.ShapeDtypeStruct((NSLAB, 128), jnp.float32),
    mesh=_MESH,
    compiler_params=pltpu.CompilerParams(
        needs_layout_passes=False, use_tc_tiling_on_sc=True),
    scratch_types=[
        pltpu.VMEM((64, 128), jnp.float32),
        pltpu.VMEM((64, 128), jnp.float32),
        pltpu.VMEM((64, 128), jnp.float32),
        pltpu.VMEM((64, 128), jnp.float32),
        pltpu.VMEM((64, 64), jnp.float32),
        pltpu.VMEM((32, 128), jnp.float32),
        pltpu.SemaphoreType.DMA,
        pltpu.SemaphoreType.DMA,
        pltpu.SemaphoreType.DMA,
        pltpu.SemaphoreType.DMA,
    ],
)
def _sc_transpose(wt_hbm, out_hbm, bin0, bin1, bout0, bout1, tin, tout,
                  si0, si1, so0, so1):
    wid = lax.axis_index("s") * 2 + lax.axis_index("c")
    bins = (bin0, bin1)
    bouts = (bout0, bout1)
    sis = (si0, si1)
    sos = (so0, so1)
    iotav = lax.iota(jnp.int32, L)

    def blk_of(k):
        return jnp.minimum(wid + k * NW, NBLK - 1)

    def issue_in(k, p):
        i0 = blk_of(k) * 128
        pltpu.async_copy(wt_hbm.at[:, pl.ds(i0, 128)], bins[p], sis[p])

    def drain_in(p):
        pltpu.make_async_copy(wt_hbm.at[:, pl.ds(0, 128)], bins[p], sis[p]).wait()

    def issue_out(k, p):
        r0 = blk_of(k) * 64
        pltpu.async_copy(bouts[p], out_hbm.at[pl.ds(r0, 64), :], sos[p])

    def drain_out(p):
        pltpu.make_async_copy(bouts[p], out_hbm.at[pl.ds(0, 64), :], sos[p]).wait()

    def transpose_block(p):
        def body(r4, _):
            for u in range(4):
                t = ((r4 * 4 + u) + iotav) & 127
                rs = t >> 1
                low = (t & 1) << 6
                for jb in range(4):
                    rowv = iotav + (jb * L)
                    v = plsc.load_gather(bins[p], [rowv, t])
                    cs = low + rowv
                    plsc.store_scatter(bouts[p], [rs, cs], v)
            return 0
        lax.fori_loop(0, 32, body, 0)

    issue_in(0, 0)
    issue_in(1, 1)

    def outer(k2, _):
        for p in (0, 1):
            k = k2 * 2 + p
            drain_in(p)

            @pl.when(k >= 2)
            def _():
                drain_out(p)

            transpose_block(p)
            issue_out(k, p)
            issue_in(k + 2, p)
        return 0

    lax.fori_loop(0, 122, outer, 0)

    drain_in(0)
    drain_out(0)

    @pl.when(wid <= 3)
    def _():
        transpose_block(0)
        issue_out(244, 0)

    drain_in(1)
    drain_out(1)

    @pl.when(wid <= 3)
    def _():
        drain_out(0)

    @pl.when(wid == 31)
    def _():
        pltpu.sync_copy(wt_hbm.at[:, pl.ds(NBLK * 128, 64)], tin)

        def body(r, _):
            t = (r + iotav) & 63
            rs = t >> 1
            low = (t & 1) << 6
            for jb in range(4):
                rowv = iotav + (jb * L)
                v = plsc.load_gather(tin, [rowv, t])
                cs = low + rowv
                plsc.store_scatter(tout, [rs, cs], v)
            return 0
        lax.fori_loop(0, 64, body, 0)
        pltpu.sync_copy(tout, out_hbm.at[pl.ds(NBLK * 64, 32), :])


@functools.partial(
    pl.kernel,
    out_type=jax.ShapeDtypeStruct((NW, L), jnp.float32),
    mesh=_MESH,
    compiler_params=pltpu.CompilerParams(
        needs_layout_passes=False, use_tc_tiling_on_sc=False),
    scratch_types=[
        pltpu.VMEM(((NB * NNEG) // 128, 128), jnp.int32),
        pltpu.VMEM((NB // 128, 128), jnp.int32),
        pltpu.VMEM((NB, D), jnp.float32),
        pltpu.VMEM((ROWS, D), jnp.float32),
        pltpu.VMEM((ROWS, D), jnp.float32),
        pltpu.VMEM((L,), jnp.float32),
        pltpu.SemaphoreType.DMA,
        pltpu.SemaphoreType.DMA,
    ],
)
def _sc_loss(h_hbm, tails_hbm, neg_hbm, w_hbm, out_hbm,
             idx_v, tails_v, h_v, rows0, rows1, out_v, sem0, sem1):
    wid = lax.axis_index("s") * 2 + lax.axis_index("c")

    pltpu.sync_copy(neg_hbm.at[wid], idx_v)
    pltpu.sync_copy(tails_hbm.at[wid], tails_v)
    pltpu.sync_copy(h_hbm.at[pl.ds(wid * NB, NB)], h_v)

    rows = (rows0, rows1)
    sems = (sem0, sem1)

    def issue(ci, p):
        r0 = ci * 2
        pltpu.async_copy(w_hbm.at[idx_v.at[r0]], rows[p].at[pl.ds(0, 128)], sems[p])
        pltpu.async_copy(w_hbm.at[idx_v.at[r0 + 1]], rows[p].at[pl.ds(128, 128)], sems[p])

    def drain(p):
        pltpu.make_async_copy(w_hbm.at[pl.ds(0, ROWS)], rows[p], sems[p]).wait()

    issue(0, 0)
    issue(1, 1)

    iotav = lax.iota(jnp.int32, L)
    rowbase = [iotav + (b * NNEG + g * L) for b in range(CB) for g in range(4)]
    zero16 = jnp.zeros((L,), jnp.float32)

    def chunk_compute(c, p, loss):
        def dbody(dd, accs):
            col = (iotav + dd) & (D - 1)
            new = []
            k = 0
            for b in range(CB):
                hrow = jnp.full((L,), c * CB + b, jnp.int32)
                hd = plsc.load_gather(h_v, [hrow, col])
                for _g in range(4):
                    wv = plsc.load_gather(rows[p], [rowbase[k], col])
                    new.append(accs[k] + wv * hd)
                    k += 1
            return tuple(new)
        accs = lax.fori_loop(0, D, dbody, tuple(zero16 for _ in range(L)))
        for a in accs:
            loss = loss + _logsig(-a)
        return loss

    def outer(c2, loss):
        for p in (0, 1):
            c = c2 * 2 + p
            drain(p)
            loss = chunk_compute(c, p, loss)
            nc = c + 2
            nc = jnp.where(nc >= NCH, nc - NCH, nc)
            issue(nc, p)
        return loss

    loss = lax.fori_loop(0, NCH // 2, outer, zero16)
    drain(0)
    drain(1)

    for r in range(2):
        pltpu.async_copy(w_hbm.at[tails_v.at[r]], rows0.at[pl.ds(r * 128, 128)], sem0)
        pltpu.async_copy(w_hbm.at[tails_v.at[2 + r]], rows1.at[pl.ds(r * 128, 128)], sem1)
    drain(0)
    drain(1)

    def pos_phase(bufref, base_off, loss):
        def bgbody(bg, loss):
            rbase = iotav + bg * L
            def dbody(dd, acc):
                col = (iotav + dd) & (D - 1)
                wv = plsc.load_gather(bufref, [rbase, col])
                hv = plsc.load_gather(h_v, [rbase + base_off, col])
                return acc + wv * hv
            s = lax.fori_loop(0, D, dbody, zero16)
            return loss + _logsig(s)
        return lax.fori_loop(0, L, bgbody, loss)

    loss = pos_phase(rows0, 0, loss)
    loss = pos_phase(rows1, 256, loss)

    out_v[...] = loss
    pltpu.sync_copy(out_v, out_hbm.at[wid])


def kernel(heads, head_embeds, tails, weights):
    del heads
    neg = _neg_idx3d()
    tails3 = tails.astype(jnp.int32).reshape(NW, NB // 128, 128)
    wlin = _sc_transpose(weights.T).reshape(NGENES, D)
    part = _sc_loss(head_embeds, tails3, neg, wlin)
    return -(jnp.sum(part) / B)


# speedup vs baseline: 2.1175x; 1.5808x over previous
import functools

import jax
import jax.numpy as jnp
import numpy as np
from jax import lax
from jax.experimental import pallas as pl
from jax.experimental.pallas import tpu as pltpu
from jax.experimental.pallas import tpu_sc as plsc

B = 16384
D = 64
NNEG = 64
NGENES = 1000000
NW = 32
NB = B // NW
CB = 4
NCH = NB // CB
ROWS = CB * NNEG
L = 16
NBLK = NGENES // 128
NSLAB = NGENES // 2

_cache = {}
_MESH = plsc.VectorSubcoreMesh(core_axis_name="c", subcore_axis_name="s")


def _neg_idx3d():
    if "neg" not in _cache:
        try:
            with jax.ensure_compile_time_eval(), \
                    jax.default_device(jax.devices("cpu")[0]):
                nt = jax.random.randint(jax.random.key(42), (B, NNEG), 0, NGENES)
                nt = np.asarray(nt, np.int32).reshape(NW, (NB * NNEG) // 128, 128)
            _cache["neg"] = nt
        except Exception:
            nt = jax.random.randint(jax.random.key(42), (B, NNEG), 0, NGENES)
            return jnp.asarray(nt, jnp.int32).reshape(NW, (NB * NNEG) // 128, 128)
    return _cache["neg"]


def _logsig(s):
    u = jnp.exp(-jnp.abs(s))
    t = u / (u + 2.0)
    t2 = t * t
    p = 1.0 + t2 * (1.0 / 3.0 + t2 * (0.2 + t2 * (1.0 / 7.0)))
    return jnp.minimum(s, 0.0) - 2.0 * t * p


@functools.partial(
    pl.kernel,
    out_type=jax.ShapeDtypeStruct((NSLAB, 128), jnp.float32),
    mesh=_MESH,
    compiler_params=pltpu.CompilerParams(
        needs_layout_passes=False, use_tc_tiling_on_sc=True),
    scratch_types=[
        pltpu.VMEM((64, 128), jnp.float32),
        pltpu.VMEM((64, 128), jnp.float32),
        pltpu.VMEM((64, 128), jnp.float32),
        pltpu.VMEM((64, 128), jnp.float32),
        pltpu.VMEM((64, 64), jnp.float32),
        pltpu.VMEM((32, 128), jnp.float32),
        pltpu.SemaphoreType.DMA,
        pltpu.SemaphoreType.DMA,
        pltpu.SemaphoreType.DMA,
        pltpu.SemaphoreType.DMA,
    ],
)
def _sc_transpose(wt_hbm, out_hbm, bin0, bin1, bout0, bout1, tin, tout,
                  si0, si1, so0, so1):
    wid = lax.axis_index("s") * 2 + lax.axis_index("c")
    bins = (bin0, bin1)
    bouts = (bout0, bout1)
    sis = (si0, si1)
    sos = (so0, so1)
    iotav = lax.iota(jnp.int32, L)

    def blk_of(k):
        return jnp.minimum(wid + k * NW, NBLK - 1)

    def issue_in(k, p):
        i0 = blk_of(k) * 128
        pltpu.async_copy(wt_hbm.at[:, pl.ds(i0, 128)], bins[p], sis[p])

    def drain_in(p):
        pltpu.make_async_copy(wt_hbm.at[:, pl.ds(0, 128)], bins[p], sis[p]).wait()

    def issue_out(k, p):
        r0 = blk_of(k) * 64
        pltpu.async_copy(bouts[p], out_hbm.at[pl.ds(r0, 64), :], sos[p])

    def drain_out(p):
        pltpu.make_async_copy(bouts[p], out_hbm.at[pl.ds(0, 64), :], sos[p]).wait()

    def transpose_block(p):
        def body(r4, _):
            gathered = []
            for u in range(4):
                t = ((r4 * 4 + u) + iotav) & 127
                rs = t >> 1
                low = (t & 1) << 6
                for jb in range(4):
                    rowv = iotav + (jb * L)
                    v = plsc.load_gather(bins[p], [rowv, t])
                    gathered.append((rs, low + rowv, v))
            for rs, cs, v in gathered:
                plsc.store_scatter(bouts[p], [rs, cs], v)
            return 0
        lax.fori_loop(0, 32, body, 0)

    issue_in(0, 0)
    issue_in(1, 1)

    def outer(k2, _):
        for p in (0, 1):
            k = k2 * 2 + p
            drain_in(p)

            @pl.when(k >= 2)
            def _():
                drain_out(p)

            transpose_block(p)
            issue_out(k, p)
            issue_in(k + 2, p)
        return 0

    lax.fori_loop(0, 122, outer, 0)

    drain_in(0)
    drain_out(0)

    @pl.when(wid <= 3)
    def _():
        transpose_block(0)
        issue_out(244, 0)

    drain_in(1)
    drain_out(1)

    @pl.when(wid <= 3)
    def _():
        drain_out(0)

    @pl.when(wid == 31)
    def _():
        pltpu.sync_copy(wt_hbm.at[:, pl.ds(NBLK * 128, 64)], tin)

        def body(r, _):
            t = (r + iotav) & 63
            rs = t >> 1
            low = (t & 1) << 6
            for jb in range(4):
                rowv = iotav + (jb * L)
                v = plsc.load_gather(tin, [rowv, t])
                cs = low + rowv
                plsc.store_scatter(tout, [rs, cs], v)
            return 0
        lax.fori_loop(0, 64, body, 0)
        pltpu.sync_copy(tout, out_hbm.at[pl.ds(NBLK * 64, 32), :])


@functools.partial(
    pl.kernel,
    out_type=jax.ShapeDtypeStruct((NW, L), jnp.float32),
    mesh=_MESH,
    compiler_params=pltpu.CompilerParams(
        needs_layout_passes=False, use_tc_tiling_on_sc=False),
    scratch_types=[
        pltpu.VMEM(((NB * NNEG) // 128, 128), jnp.int32),
        pltpu.VMEM((NB // 128, 128), jnp.int32),
        pltpu.VMEM((NB, D), jnp.float32),
        pltpu.VMEM((ROWS, D), jnp.float32),
        pltpu.VMEM((ROWS, D), jnp.float32),
        pltpu.VMEM((L,), jnp.float32),
        pltpu.SemaphoreType.DMA,
        pltpu.SemaphoreType.DMA,
    ],
)
def _sc_loss(h_hbm, tails_hbm, neg_hbm, w_hbm, out_hbm,
             idx_v, tails_v, h_v, rows0, rows1, out_v, sem0, sem1):
    wid = lax.axis_index("s") * 2 + lax.axis_index("c")

    pltpu.sync_copy(neg_hbm.at[wid], idx_v)
    pltpu.sync_copy(tails_hbm.at[wid], tails_v)
    pltpu.sync_copy(h_hbm.at[pl.ds(wid * NB, NB)], h_v)

    rows = (rows0, rows1)
    sems = (sem0, sem1)

    def issue(ci, p):
        r0 = ci * 2
        pltpu.async_copy(w_hbm.at[idx_v.at[r0]], rows[p].at[pl.ds(0, 128)], sems[p])
        pltpu.async_copy(w_hbm.at[idx_v.at[r0 + 1]], rows[p].at[pl.ds(128, 128)], sems[p])

    def drain(p):
        pltpu.make_async_copy(w_hbm.at[pl.ds(0, ROWS)], rows[p], sems[p]).wait()

    issue(0, 0)
    issue(1, 1)

    iotav = lax.iota(jnp.int32, L)
    rowbase = [iotav + (b * NNEG + g * L) for b in range(CB) for g in range(4)]
    zero16 = jnp.zeros((L,), jnp.float32)

    def chunk_compute(c, p, loss):
        def dbody(dd, accs):
            col = (iotav + dd) & (D - 1)
            new = []
            k = 0
            for b in range(CB):
                hrow = jnp.full((L,), c * CB + b, jnp.int32)
                hd = plsc.load_gather(h_v, [hrow, col])
                for _g in range(4):
                    wv = plsc.load_gather(rows[p], [rowbase[k], col])
                    new.append(accs[k] + wv * hd)
                    k += 1
            return tuple(new)
        accs = lax.fori_loop(0, D, dbody, tuple(zero16 for _ in range(L)))
        for a in accs:
            loss = loss + _logsig(-a)
        return loss

    def outer(c2, loss):
        for p in (0, 1):
            c = c2 * 2 + p
            drain(p)
            loss = chunk_compute(c, p, loss)
            nc = c + 2
            nc = jnp.where(nc >= NCH, nc - NCH, nc)
            issue(nc, p)
        return loss

    loss = lax.fori_loop(0, NCH // 2, outer, zero16)
    drain(0)
    drain(1)

    for r in range(2):
        pltpu.async_copy(w_hbm.at[tails_v.at[r]], rows0.at[pl.ds(r * 128, 128)], sem0)
        pltpu.async_copy(w_hbm.at[tails_v.at[2 + r]], rows1.at[pl.ds(r * 128, 128)], sem1)
    drain(0)
    drain(1)

    def pos_phase(bufref, base_off, loss):
        def bgbody(bg, loss):
            rbase = iotav + bg * L
            def dbody(dd, acc):
                col = (iotav + dd) & (D - 1)
                wv = plsc.load_gather(bufref, [rbase, col])
                hv = plsc.load_gather(h_v, [rbase + base_off, col])
                return acc + wv * hv
            s = lax.fori_loop(0, D, dbody, zero16)
            return loss + _logsig(s)
        return lax.fori_loop(0, L, bgbody, loss)

    loss = pos_phase(rows0, 0, loss)
    loss = pos_phase(rows1, 256, loss)

    out_v[...] = loss
    pltpu.sync_copy(out_v, out_hbm.at[wid])


def kernel(heads, head_embeds, tails, weights):
    del heads
    neg = _neg_idx3d()
    tails3 = tails.astype(jnp.int32).reshape(NW, NB // 128, 128)
    wlin = _sc_transpose(weights.T).reshape(NGENES, D)
    part = _sc_loss(head_embeds, tails3, neg, wlin)
    return -(jnp.sum(part) / B)


# speedup vs baseline: 2.2039x; 1.0408x over previous
import functools

import jax
import jax.numpy as jnp
import numpy as np
from jax import lax
from jax.experimental import pallas as pl
from jax.experimental.pallas import tpu as pltpu
from jax.experimental.pallas import tpu_sc as plsc

B = 16384
D = 64
NNEG = 64
NGENES = 1000000
NW = 32
NB = B // NW
CB = 4
NCH = NB // CB
ROWS = CB * NNEG
L = 16
NBLK = NGENES // 128
NSLAB = NGENES // 2

_cache = {}
_MESH = plsc.VectorSubcoreMesh(core_axis_name="c", subcore_axis_name="s")


def _neg_idx3d():
    if "neg" not in _cache:
        try:
            with jax.ensure_compile_time_eval(), \
                    jax.default_device(jax.devices("cpu")[0]):
                nt = jax.random.randint(jax.random.key(42), (B, NNEG), 0, NGENES)
                nt = np.asarray(nt, np.int32).reshape(NW, (NB * NNEG) // 128, 128)
            _cache["neg"] = nt
        except Exception:
            nt = jax.random.randint(jax.random.key(42), (B, NNEG), 0, NGENES)
            return jnp.asarray(nt, jnp.int32).reshape(NW, (NB * NNEG) // 128, 128)
    return _cache["neg"]


def _logsig(s):
    u = jnp.exp(-jnp.abs(s))
    t = u / (u + 2.0)
    t2 = t * t
    p = 1.0 + t2 * (1.0 / 3.0 + t2 * (0.2 + t2 * (1.0 / 7.0)))
    return jnp.minimum(s, 0.0) - 2.0 * t * p


@functools.partial(
    pl.kernel,
    out_type=jax.ShapeDtypeStruct((NSLAB, 128), jnp.float32),
    mesh=_MESH,
    compiler_params=pltpu.CompilerParams(
        needs_layout_passes=False, use_tc_tiling_on_sc=True),
    scratch_types=[
        pltpu.VMEM((64, 128), jnp.float32),
        pltpu.VMEM((64, 128), jnp.float32),
        pltpu.VMEM((64, 128), jnp.float32),
        pltpu.VMEM((64, 128), jnp.float32),
        pltpu.VMEM((64, 64), jnp.float32),
        pltpu.VMEM((32, 128), jnp.float32),
        pltpu.SemaphoreType.DMA,
        pltpu.SemaphoreType.DMA,
        pltpu.SemaphoreType.DMA,
        pltpu.SemaphoreType.DMA,
    ],
)
def _sc_transpose(wt_hbm, out_hbm, bin0, bin1, bout0, bout1, tin, tout,
                  si0, si1, so0, so1):
    wid = lax.axis_index("s") * 2 + lax.axis_index("c")
    bins = (bin0, bin1)
    bouts = (bout0, bout1)
    sis = (si0, si1)
    sos = (so0, so1)
    iotav = lax.iota(jnp.int32, L)

    def blk_of(k):
        return jnp.minimum(wid + k * NW, NBLK - 1)

    def issue_in(k, p):
        i0 = blk_of(k) * 128
        pltpu.async_copy(wt_hbm.at[:, pl.ds(i0, 128)], bins[p], sis[p])

    def drain_in(p):
        pltpu.make_async_copy(wt_hbm.at[:, pl.ds(0, 128)], bins[p], sis[p]).wait()

    def issue_out(k, p):
        r0 = blk_of(k) * 64
        pltpu.async_copy(bouts[p], out_hbm.at[pl.ds(r0, 64), :], sos[p])

    def drain_out(p):
        pltpu.make_async_copy(bouts[p], out_hbm.at[pl.ds(0, 64), :], sos[p]).wait()

    def transpose_block(p):
        def body(r4, _):
            gathered = []
            for u in range(4):
                t = ((r4 * 4 + u) + iotav) & 127
                rs = t >> 1
                low = (t & 1) << 6
                for jb in range(4):
                    rowv = iotav + (jb * L)
                    v = plsc.load_gather(bins[p], [rowv, t])
                    gathered.append((rs, low + rowv, v))
            for rs, cs, v in gathered:
                plsc.store_scatter(bouts[p], [rs, cs], v)
            return 0
        lax.fori_loop(0, 32, body, 0)

    issue_in(0, 0)
    issue_in(1, 1)

    def outer(k2, _):
        for p in (0, 1):
            k = k2 * 2 + p
            drain_in(p)

            @pl.when(k >= 2)
            def _():
                drain_out(p)

            transpose_block(p)
            issue_out(k, p)
            issue_in(k + 2, p)
        return 0

    lax.fori_loop(0, 122, outer, 0)

    drain_in(0)
    drain_out(0)

    @pl.when(wid <= 3)
    def _():
        transpose_block(0)
        issue_out(244, 0)

    drain_in(1)
    drain_out(1)

    @pl.when(wid <= 3)
    def _():
        drain_out(0)

    @pl.when(wid == 31)
    def _():
        pltpu.sync_copy(wt_hbm.at[:, pl.ds(NBLK * 128, 64)], tin)

        def body(r, _):
            t = (r + iotav) & 63
            rs = t >> 1
            low = (t & 1) << 6
            for jb in range(4):
                rowv = iotav + (jb * L)
                v = plsc.load_gather(tin, [rowv, t])
                cs = low + rowv
                plsc.store_scatter(tout, [rs, cs], v)
            return 0
        lax.fori_loop(0, 64, body, 0)
        pltpu.sync_copy(tout, out_hbm.at[pl.ds(NBLK * 64, 32), :])


@functools.partial(
    pl.kernel,
    out_type=jax.ShapeDtypeStruct((NW, L), jnp.float32),
    mesh=_MESH,
    compiler_params=pltpu.CompilerParams(
        needs_layout_passes=False, use_tc_tiling_on_sc=False),
    scratch_types=[
        pltpu.VMEM(((NB * NNEG) // 128, 128), jnp.int32),
        pltpu.VMEM((NB // 128, 128), jnp.int32),
        pltpu.VMEM((NB, D), jnp.float32),
        pltpu.VMEM((ROWS, D), jnp.float32),
        pltpu.VMEM((ROWS, D), jnp.float32),
        pltpu.VMEM((ROWS, D), jnp.float32),
        pltpu.VMEM((L,), jnp.float32),
        pltpu.SemaphoreType.DMA,
        pltpu.SemaphoreType.DMA,
        pltpu.SemaphoreType.DMA,
    ],
)
def _sc_loss(h_hbm, tails_hbm, neg_hbm, w_hbm, out_hbm,
             idx_v, tails_v, h_v, rows0, rows1, rows2, out_v,
             sem0, sem1, sem2):
    wid = lax.axis_index("s") * 2 + lax.axis_index("c")

    pltpu.sync_copy(neg_hbm.at[wid], idx_v)
    pltpu.sync_copy(tails_hbm.at[wid], tails_v)
    pltpu.sync_copy(h_hbm.at[pl.ds(wid * NB, NB)], h_v)

    rows = (rows0, rows1, rows2)
    sems = (sem0, sem1, sem2)

    def issue(ci, p):
        r0 = ci * 2
        pltpu.async_copy(w_hbm.at[idx_v.at[r0]], rows[p].at[pl.ds(0, 128)], sems[p])
        pltpu.async_copy(w_hbm.at[idx_v.at[r0 + 1]], rows[p].at[pl.ds(128, 128)], sems[p])

    def drain(p):
        pltpu.make_async_copy(w_hbm.at[pl.ds(0, ROWS)], rows[p], sems[p]).wait()

    issue(0, 0)
    issue(1, 1)
    issue(2, 2)

    iotav = lax.iota(jnp.int32, L)
    rowbase = [iotav + (b * NNEG + g * L) for b in range(CB) for g in range(4)]
    zero16 = jnp.zeros((L,), jnp.float32)

    def chunk_compute(c, p, loss):
        def dbody(dd, accs):
            col = (iotav + dd) & (D - 1)
            new = []
            k = 0
            for b in range(CB):
                hrow = jnp.full((L,), c * CB + b, jnp.int32)
                hd = plsc.load_gather(h_v, [hrow, col])
                for _g in range(4):
                    wv = plsc.load_gather(rows[p], [rowbase[k], col])
                    new.append(accs[k] + wv * hd)
                    k += 1
            return tuple(new)
        accs = lax.fori_loop(0, D, dbody, tuple(zero16 for _ in range(L)))
        for a in accs:
            loss = loss + _logsig(-a)
        return loss

    def process(c, p, loss):
        drain(p)
        loss = chunk_compute(c, p, loss)
        nc = c + 3
        nc = jnp.where(nc >= NCH, nc - NCH, nc)
        issue(nc, p)
        return loss

    def outer(c3, loss):
        for p in (0, 1, 2):
            loss = process(c3 * 3 + p, p, loss)
        return loss

    loss = lax.fori_loop(0, NCH // 3, outer, zero16)
    loss = process(NCH - 2, 0, loss)
    loss = process(NCH - 1, 1, loss)
    drain(2)
    drain(0)
    drain(1)

    for r in range(2):
        pltpu.async_copy(w_hbm.at[tails_v.at[r]], rows0.at[pl.ds(r * 128, 128)], sem0)
        pltpu.async_copy(w_hbm.at[tails_v.at[2 + r]], rows1.at[pl.ds(r * 128, 128)], sem1)
    drain(0)
    drain(1)

    def pos_phase(bufref, base_off, loss):
        def bgbody(bg, loss):
            rbase = iotav + bg * L
            def dbody(dd, acc):
                col = (iotav + dd) & (D - 1)
                wv = plsc.load_gather(bufref, [rbase, col])
                hv = plsc.load_gather(h_v, [rbase + base_off, col])
                return acc + wv * hv
            s = lax.fori_loop(0, D, dbody, zero16)
            return loss + _logsig(s)
        return lax.fori_loop(0, L, bgbody, loss)

    loss = pos_phase(rows0, 0, loss)
    loss = pos_phase(rows1, 256, loss)

    out_v[...] = loss
    pltpu.sync_copy(out_v, out_hbm.at[wid])


def kernel(heads, head_embeds, tails, weights):
    del heads
    neg = _neg_idx3d()
    tails3 = tails.astype(jnp.int32).reshape(NW, NB // 128, 128)
    wlin = _sc_transpose(weights.T).reshape(NGENES, D)
    part = _sc_loss(head_embeds, tails3, neg, wlin)
    return -(jnp.sum(part) / B)


# speedup vs baseline: 2.4459x; 1.1098x over previous
import functools

import jax
import jax.numpy as jnp
import numpy as np
from jax import lax
from jax.experimental import pallas as pl
from jax.experimental.pallas import tpu as pltpu
from jax.experimental.pallas import tpu_sc as plsc

B = 16384
D = 64
NNEG = 64
NGENES = 1000000
NW = 32
NB = B // NW
CB = 4
NCH = NB // CB
ROWS = CB * NNEG
L = 16
NBLK = NGENES // 128
NSLAB = NGENES // 2

_cache = {}
_MESH = plsc.VectorSubcoreMesh(core_axis_name="c", subcore_axis_name="s")


def _neg_idx3d():
    if "neg" not in _cache:
        try:
            with jax.ensure_compile_time_eval(), \
                    jax.default_device(jax.devices("cpu")[0]):
                nt = jax.random.randint(jax.random.key(42), (B, NNEG), 0, NGENES)
                nt = np.asarray(nt, np.int32).reshape(NW, (NB * NNEG) // 128, 128)
            _cache["neg"] = nt
        except Exception:
            nt = jax.random.randint(jax.random.key(42), (B, NNEG), 0, NGENES)
            return jnp.asarray(nt, jnp.int32).reshape(NW, (NB * NNEG) // 128, 128)
    return _cache["neg"]


def _logsig(s):
    u = jnp.exp(-jnp.abs(s))
    t = u / (u + 2.0)
    t2 = t * t
    p = 1.0 + t2 * (1.0 / 3.0 + t2 * (0.2 + t2 * (1.0 / 7.0)))
    return jnp.minimum(s, 0.0) - 2.0 * t * p


@functools.partial(
    pl.kernel,
    out_type=jax.ShapeDtypeStruct((NSLAB, 128), jnp.float32),
    mesh=_MESH,
    compiler_params=pltpu.CompilerParams(
        needs_layout_passes=False, use_tc_tiling_on_sc=True),
    scratch_types=[
        pltpu.VMEM((64, 128), jnp.float32),
        pltpu.VMEM((64, 128), jnp.float32),
        pltpu.VMEM((64, 128), jnp.float32),
        pltpu.VMEM((64, 128), jnp.float32),
        pltpu.VMEM((64, 128), jnp.float32),
        pltpu.VMEM((64, 128), jnp.float32),
        pltpu.VMEM((64, 64), jnp.float32),
        pltpu.VMEM((32, 128), jnp.float32),
        pltpu.SemaphoreType.DMA,
        pltpu.SemaphoreType.DMA,
        pltpu.SemaphoreType.DMA,
        pltpu.SemaphoreType.DMA,
        pltpu.SemaphoreType.DMA,
        pltpu.SemaphoreType.DMA,
    ],
)
def _sc_transpose(wt_hbm, out_hbm, bin0, bin1, bin2, bout0, bout1, bout2,
                  tin, tout, si0, si1, si2, so0, so1, so2):
    wid = lax.axis_index("s") * 2 + lax.axis_index("c")
    bins = (bin0, bin1, bin2)
    bouts = (bout0, bout1, bout2)
    sis = (si0, si1, si2)
    sos = (so0, so1, so2)
    iotav = lax.iota(jnp.int32, L)

    def blk_of(k):
        return jnp.minimum(wid + k * NW, NBLK - 1)

    def issue_in(k, p):
        i0 = blk_of(k) * 128
        pltpu.async_copy(wt_hbm.at[:, pl.ds(i0, 128)], bins[p], sis[p])

    def drain_in(p):
        pltpu.make_async_copy(wt_hbm.at[:, pl.ds(0, 128)], bins[p], sis[p]).wait()

    def issue_out(k, p):
        r0 = blk_of(k) * 64
        pltpu.async_copy(bouts[p], out_hbm.at[pl.ds(r0, 64), :], sos[p])

    def drain_out(p):
        pltpu.make_async_copy(bouts[p], out_hbm.at[pl.ds(0, 64), :], sos[p]).wait()

    def transpose_block(p):
        def body(r4, _):
            gathered = []
            for u in range(4):
                t = ((r4 * 4 + u) + iotav) & 127
                rs = t >> 1
                low = (t & 1) << 6
                for jb in range(4):
                    rowv = iotav + (jb * L)
                    v = plsc.load_gather(bins[p], [rowv, t])
                    gathered.append((rs, low + rowv, v))
            for rs, cs, v in gathered:
                plsc.store_scatter(bouts[p], [rs, cs], v)
            return 0
        lax.fori_loop(0, 32, body, 0)

    issue_in(0, 0)
    issue_in(1, 1)
    issue_in(2, 2)

    def process(k, p):
        kk = jnp.asarray(k, jnp.int32)
        drain_in(p)

        @pl.when(kk >= 3)
        def _():
            drain_out(p)

        transpose_block(p)
        issue_out(k, p)
        issue_in(k + 3, p)

    def outer(k3, _):
        for p in (0, 1, 2):
            process(k3 * 3 + p, p)
        return 0

    lax.fori_loop(0, 81, outer, 0)
    process(243, 0)

    drain_in(1)
    drain_out(1)

    @pl.when(wid <= 3)
    def _():
        transpose_block(1)
        issue_out(244, 1)

    drain_in(2)
    drain_out(2)
    drain_in(0)
    drain_out(0)

    @pl.when(wid <= 3)
    def _():
        drain_out(1)

    @pl.when(wid == 31)
    def _():
        pltpu.sync_copy(wt_hbm.at[:, pl.ds(NBLK * 128, 64)], tin)

        def body(r, _):
            t = (r + iotav) & 63
            rs = t >> 1
            low = (t & 1) << 6
            for jb in range(4):
                rowv = iotav + (jb * L)
                v = plsc.load_gather(tin, [rowv, t])
                cs = low + rowv
                plsc.store_scatter(tout, [rs, cs], v)
            return 0
        lax.fori_loop(0, 64, body, 0)
        pltpu.sync_copy(tout, out_hbm.at[pl.ds(NBLK * 64, 32), :])


@functools.partial(
    pl.kernel,
    out_type=jax.ShapeDtypeStruct((NW, L), jnp.float32),
    mesh=_MESH,
    compiler_params=pltpu.CompilerParams(
        needs_layout_passes=False, use_tc_tiling_on_sc=False),
    scratch_types=[
        pltpu.VMEM(((NB * NNEG) // 128, 128), jnp.int32),
        pltpu.VMEM((NB // 128, 128), jnp.int32),
        pltpu.VMEM((NB, D), jnp.float32),
        pltpu.VMEM((ROWS, D), jnp.float32),
        pltpu.VMEM((ROWS, D), jnp.float32),
        pltpu.VMEM((ROWS, D), jnp.float32),
        pltpu.VMEM((L,), jnp.float32),
        pltpu.SemaphoreType.DMA,
        pltpu.SemaphoreType.DMA,
        pltpu.SemaphoreType.DMA,
    ],
)
def _sc_loss(h_hbm, tails_hbm, neg_hbm, w_hbm, out_hbm,
             idx_v, tails_v, h_v, rows0, rows1, rows2, out_v,
             sem0, sem1, sem2):
    wid = lax.axis_index("s") * 2 + lax.axis_index("c")

    pltpu.sync_copy(neg_hbm.at[wid], idx_v)
    pltpu.sync_copy(tails_hbm.at[wid], tails_v)
    pltpu.sync_copy(h_hbm.at[pl.ds(wid * NB, NB)], h_v)

    rows = (rows0, rows1, rows2)
    sems = (sem0, sem1, sem2)

    def issue(ci, p):
        r0 = ci * 2
        pltpu.async_copy(w_hbm.at[idx_v.at[r0]], rows[p].at[pl.ds(0, 128)], sems[p])
        pltpu.async_copy(w_hbm.at[idx_v.at[r0 + 1]], rows[p].at[pl.ds(128, 128)], sems[p])

    def drain(p):
        pltpu.make_async_copy(w_hbm.at[pl.ds(0, ROWS)], rows[p], sems[p]).wait()

    issue(0, 0)
    issue(1, 1)
    issue(2, 2)

    iotav = lax.iota(jnp.int32, L)
    rowbase = [iotav + (b * NNEG + g * L) for b in range(CB) for g in range(4)]
    zero16 = jnp.zeros((L,), jnp.float32)

    def chunk_compute(c, p, loss):
        def dbody(dd, accs):
            col = (iotav + dd) & (D - 1)
            new = []
            k = 0
            for b in range(CB):
                hrow = jnp.full((L,), c * CB + b, jnp.int32)
                hd = plsc.load_gather(h_v, [hrow, col])
                for _g in range(4):
                    wv = plsc.load_gather(rows[p], [rowbase[k], col])
                    new.append(accs[k] + wv * hd)
                    k += 1
            return tuple(new)
        accs = lax.fori_loop(0, D, dbody, tuple(zero16 for _ in range(L)))
        for a in accs:
            loss = loss + _logsig(-a)
        return loss

    def process(c, p, loss):
        drain(p)
        loss = chunk_compute(c, p, loss)
        nc = c + 3
        nc = jnp.where(nc >= NCH, nc - NCH, nc)
        issue(nc, p)
        return loss

    def outer(c3, loss):
        for p in (0, 1, 2):
            loss = process(c3 * 3 + p, p, loss)
        return loss

    loss = lax.fori_loop(0, NCH // 3, outer, zero16)
    loss = process(NCH - 2, 0, loss)
    loss = process(NCH - 1, 1, loss)
    drain(2)
    drain(0)
    drain(1)

    for r in range(2):
        pltpu.async_copy(w_hbm.at[tails_v.at[r]], rows0.at[pl.ds(r * 128, 128)], sem0)
        pltpu.async_copy(w_hbm.at[tails_v.at[2 + r]], rows1.at[pl.ds(r * 128, 128)], sem1)
    drain(0)
    drain(1)

    def pos_phase(bufref, base_off, loss):
        def bgbody(bg, loss):
            rbase = iotav + bg * L
            def dbody(dd, acc):
                col = (iotav + dd) & (D - 1)
                wv = plsc.load_gather(bufref, [rbase, col])
                hv = plsc.load_gather(h_v, [rbase + base_off, col])
                return acc + wv * hv
            s = lax.fori_loop(0, D, dbody, zero16)
            return loss + _logsig(s)
        return lax.fori_loop(0, L, bgbody, loss)

    loss = pos_phase(rows0, 0, loss)
    loss = pos_phase(rows1, 256, loss)

    out_v[...] = loss
    pltpu.sync_copy(out_v, out_hbm.at[wid])


def kernel(heads, head_embeds, tails, weights):
    del heads
    neg = _neg_idx3d()
    tails3 = tails.astype(jnp.int32).reshape(NW, NB // 128, 128)
    wlin = _sc_transpose(weights.T).reshape(NGENES, D)
    part = _sc_loss(head_embeds, tails3, neg, wlin)
    return -(jnp.sum(part) / B)
